# baseline (speedup 1.0000x reference)
import functools

import jax
import jax.numpy as jnp
from jax import lax
from jax.experimental import pallas as pl
from jax.experimental.pallas import tpu as pltpu
from jax.experimental.pallas import tpu_sc as plsc

SILU_GAIN = 1.6765
_NC = 2
_NS = 16


def _act(x):
    return jax.nn.silu(x) * SILU_GAIN


def _sc_charge_gather(charges, senders, receivers):
    N = charges.shape[0]
    E = senders.shape[0]
    nw = _NC * _NS
    per = E // nw
    L = 16
    assert E % nw == 0 and per % L == 0

    mesh = plsc.VectorSubcoreMesh(
        core_axis_name="c", subcore_axis_name="s",
        num_cores=_NC, num_subcores=_NS)

    @functools.partial(
        pl.kernel,
        out_type=(jax.ShapeDtypeStruct((E,), jnp.int32),
                  jax.ShapeDtypeStruct((E,), jnp.int32)),
        mesh=mesh,
        compiler_params=pltpu.CompilerParams(needs_layout_passes=False),
        scratch_types=[
            pltpu.VMEM((N,), jnp.int32),
            pltpu.VMEM((per,), jnp.int32),
            pltpu.VMEM((per,), jnp.int32),
            pltpu.VMEM((per,), jnp.int32),
            pltpu.VMEM((per,), jnp.int32),
            pltpu.SemaphoreType.DMA,
            pltpu.SemaphoreType.DMA,
            pltpu.SemaphoreType.DMA,
        ],
    )
    def k(charges_hbm, senders_hbm, receivers_hbm, cs_hbm, cr_hbm,
          table_v, idx_s, idx_r, out_s, out_r, sem_t, sem_s, sem_r):
        wid = lax.axis_index("s") * _NC + lax.axis_index("c")
        base = wid * per
        sl = pl.ds(base, per)
        cp_t = pltpu.async_copy(charges_hbm, table_v, sem_t)
        cp_s = pltpu.async_copy(senders_hbm.at[sl], idx_s, sem_s)
        cp_r = pltpu.async_copy(receivers_hbm.at[sl], idx_r, sem_r)
        cp_t.wait()

        def gather_loop(idx_v, out_v):
            def body(i, carry):
                s = pl.ds(i * L, L)
                out_v[s] = plsc.load_gather(table_v, [idx_v[s]])
                return carry
            lax.fori_loop(0, per // L, body, 0)

        cp_s.wait()
        gather_loop(idx_s, out_s)
        w_s = pltpu.async_copy(out_s, cs_hbm.at[sl], sem_s)
        cp_r.wait()
        gather_loop(idx_r, out_r)
        w_r = pltpu.async_copy(out_r, cr_hbm.at[sl], sem_r)
        w_s.wait()
        w_r.wait()

    return k(charges, senders, receivers)


def _tc_main(rbf_packed, E, n_rbf, cs, cr, emb_pad, w_s, w_r, w_rbf, b_rbf,
             w_q, b_out, block_e):
    pack = 128 // n_rbf
    emb = w_s.shape[1]
    C = emb_pad.shape[0]
    G = E // block_e
    dn_t = (((0,), (0,)), ((), ()))

    def body(cs_ref, cr_ref, rbf_ref, emb_ref, ws_ref, wr_ref, wrbf_ref,
             brbfp_ref, wq_ref, bout_ref, out_ref, tbl_ref, bdw_ref):
        @pl.when(pl.program_id(0) == 0)
        def _():
            tbl_ref[0:C, :] = jnp.dot(
                emb_ref[...], ws_ref[...],
                preferred_element_type=jnp.float32).astype(jnp.bfloat16)
            tbl_ref[C:2 * C, :] = jnp.dot(
                emb_ref[...], wr_ref[...],
                preferred_element_type=jnp.float32).astype(jnp.bfloat16)
            tbl_ref[2 * C:, :] = (wq_ref[...] * SILU_GAIN).astype(jnp.bfloat16)
            bdw_ref[...] = jnp.zeros((128, pack * emb), jnp.bfloat16)
            for j in range(pack):
                bdw_ref[pl.ds(n_rbf * j, n_rbf), pl.ds(emb * j, emb)] = (
                    wrbf_ref[...])

        off = pl.program_id(0) * block_e
        cs = cs_ref[pl.ds(off, block_e)]
        cr = cr_ref[pl.ds(off, block_e)]
        iot = lax.broadcasted_iota(jnp.int32, (block_e, C), 1)
        oh_s = (iot == cs[:, None]).astype(jnp.bfloat16)
        oh_r = (iot == cr[:, None]).astype(jnp.bfloat16)
        z1p = jnp.dot(rbf_ref[...].astype(jnp.bfloat16), bdw_ref[...],
                      preferred_element_type=jnp.float32) + brbfp_ref[...]
        r1p = jax.nn.silu(z1p).astype(jnp.bfloat16)
        r1 = r1p.reshape(block_e, emb)
        h = jnp.concatenate([oh_s, oh_r, r1], axis=1)
        out_ref[...] = _act(
            jnp.dot(h, tbl_ref[...], preferred_element_type=jnp.float32)
            + bout_ref[...])

    return pl.pallas_call(
        body,
        grid=(G,),
        in_specs=[
            pl.BlockSpec((E,), lambda i: (0,)),
            pl.BlockSpec((E,), lambda i: (0,)),
            pl.BlockSpec((block_e // pack, 128), lambda i: (i, 0)),
            pl.BlockSpec((C, emb), lambda i: (0, 0)),
            pl.BlockSpec((emb, emb), lambda i: (0, 0)),
            pl.BlockSpec((emb, emb), lambda i: (0, 0)),
            pl.BlockSpec((n_rbf, emb), lambda i: (0, 0)),
            pl.BlockSpec((1, pack * emb), lambda i: (0, 0)),
            pl.BlockSpec((emb, emb), lambda i: (0, 0)),
            pl.BlockSpec((1, emb), lambda i: (0, 0)),
        ],
        out_specs=pl.BlockSpec((block_e, emb), lambda i: (i, 0)),
        out_shape=jax.ShapeDtypeStruct((E, emb), jnp.float32),
        scratch_shapes=[
            pltpu.VMEM((2 * C + emb, emb), jnp.bfloat16),
            pltpu.VMEM((128, pack * emb), jnp.bfloat16),
        ],
    )(cs, cr, rbf_packed, emb_pad, w_s, w_r, w_rbf,
      jnp.tile(b_rbf, (1, pack)), w_q, b_out)


def kernel(rbf, charges, differences, senders, receivers,
           embed_table, W_rbf, b_rbf, W_out, b_out):
    del differences
    E, n_rbf = rbf.shape
    emb = embed_table.shape[1]

    charges = charges.astype(jnp.int32)
    senders = senders.astype(jnp.int32)
    receivers = receivers.astype(jnp.int32)

    cs, cr = _sc_charge_gather(charges, senders, receivers)

    block_e = 3200

    C = 128
    emb_pad = jnp.zeros((C, emb), jnp.float32).at[:embed_table.shape[0]].set(
        embed_table)
    w_s = W_out[:emb]
    w_r = W_out[emb:2 * emb]
    w_q = W_out[2 * emb:]

    rbf_packed = rbf.reshape(E // (128 // n_rbf), 128)
    return _tc_main(rbf_packed, E, n_rbf, cs, cr, emb_pad, w_s, w_r,
                    W_rbf.astype(jnp.bfloat16), b_rbf.reshape(1, emb),
                    w_q, b_out.reshape(1, emb), block_e)


# speedup vs baseline: 1.4444x; 1.4444x over previous
import functools

import jax
import jax.numpy as jnp
from jax import lax
from jax.experimental import pallas as pl
from jax.experimental.pallas import tpu as pltpu
from jax.experimental.pallas import tpu_sc as plsc

SILU_GAIN = 1.6765
_NC = 2
_NS = 16


def _act(x):
    return jax.nn.silu(x) * SILU_GAIN


def _sc_charge_gather(charges, senders, receivers):
    N = charges.shape[0]
    E = senders.shape[0]
    nw = _NC * _NS
    per = E // nw
    L = 16
    assert E % nw == 0 and per % L == 0

    mesh = plsc.VectorSubcoreMesh(
        core_axis_name="c", subcore_axis_name="s",
        num_cores=_NC, num_subcores=_NS)

    @functools.partial(
        pl.kernel,
        out_type=(jax.ShapeDtypeStruct((E,), jnp.int32),
                  jax.ShapeDtypeStruct((E,), jnp.int32)),
        mesh=mesh,
        compiler_params=pltpu.CompilerParams(needs_layout_passes=False),
        scratch_types=[
            pltpu.VMEM((N,), jnp.int32),
            pltpu.VMEM((per,), jnp.int32),
            pltpu.VMEM((per,), jnp.int32),
            pltpu.VMEM((per,), jnp.int32),
            pltpu.VMEM((per,), jnp.int32),
            pltpu.SemaphoreType.DMA,
            pltpu.SemaphoreType.DMA,
            pltpu.SemaphoreType.DMA,
        ],
    )
    def k(charges_hbm, senders_hbm, receivers_hbm, cs_hbm, cr_hbm,
          table_v, idx_s, idx_r, out_s, out_r, sem_t, sem_s, sem_r):
        wid = lax.axis_index("s") * _NC + lax.axis_index("c")
        base = wid * per
        sl = pl.ds(base, per)
        cp_t = pltpu.async_copy(charges_hbm, table_v, sem_t)
        cp_s = pltpu.async_copy(senders_hbm.at[sl], idx_s, sem_s)
        cp_r = pltpu.async_copy(receivers_hbm.at[sl], idx_r, sem_r)
        cp_t.wait()

        def gather_loop(idx_v, out_v):
            def body(i, carry):
                s = pl.ds(i * L, L)
                out_v[s] = plsc.load_gather(table_v, [idx_v[s]])
                return carry
            lax.fori_loop(0, per // L, body, 0)

        cp_s.wait()
        gather_loop(idx_s, out_s)
        w_s = pltpu.async_copy(out_s, cs_hbm.at[sl], sem_s)
        cp_r.wait()
        gather_loop(idx_r, out_r)
        w_r = pltpu.async_copy(out_r, cr_hbm.at[sl], sem_r)
        w_s.wait()
        w_r.wait()

    return k(charges, senders, receivers)


def _tc_main(rbf, cs, cr, emb_pad, w_s, w_r, w_rbf, b_rbf, w_q, b_out,
             block_e):
    E, n_rbf = rbf.shape
    emb = w_s.shape[1]
    C = emb_pad.shape[0]
    G = E // block_e
    dn_t = (((0,), (0,)), ((), ()))

    def body(cs_ref, cr_ref, rbf_ref, emb_ref, ws_ref, wr_ref, wrbf_ref,
             brbf_ref, wq_ref, bout_ref, out_ref, tbl_ref):
        @pl.when(pl.program_id(0) == 0)
        def _():
            tbl_ref[0:C, :] = jnp.dot(
                emb_ref[...], ws_ref[...],
                preferred_element_type=jnp.float32).astype(jnp.bfloat16)
            tbl_ref[C:2 * C, :] = jnp.dot(
                emb_ref[...], wr_ref[...],
                preferred_element_type=jnp.float32).astype(jnp.bfloat16)
            tbl_ref[2 * C:, :] = (wq_ref[...] * SILU_GAIN).astype(jnp.bfloat16)

        off = pl.program_id(0) * block_e
        cs = cs_ref[pl.ds(off, block_e)]
        cr = cr_ref[pl.ds(off, block_e)]
        iot = lax.broadcasted_iota(jnp.int32, (block_e, C), 1)
        oh_s = (iot == cs[:, None]).astype(jnp.bfloat16)
        oh_r = (iot == cr[:, None]).astype(jnp.bfloat16)
        z1 = jnp.dot(rbf_ref[...], wrbf_ref[...],
                     preferred_element_type=jnp.float32) + brbf_ref[...]
        r1 = jax.nn.silu(z1).astype(jnp.bfloat16)
        h = jnp.concatenate([oh_s, oh_r, r1], axis=1)
        out_ref[...] = _act(
            jnp.dot(h, tbl_ref[...], preferred_element_type=jnp.float32)
            + bout_ref[...])

    return pl.pallas_call(
        body,
        grid=(G,),
        in_specs=[
            pl.BlockSpec((E,), lambda i: (0,)),
            pl.BlockSpec((E,), lambda i: (0,)),
            pl.BlockSpec((block_e, n_rbf), lambda i: (i, 0)),
            pl.BlockSpec((C, emb), lambda i: (0, 0)),
            pl.BlockSpec((emb, emb), lambda i: (0, 0)),
            pl.BlockSpec((emb, emb), lambda i: (0, 0)),
            pl.BlockSpec((n_rbf, emb), lambda i: (0, 0)),
            pl.BlockSpec((1, emb), lambda i: (0, 0)),
            pl.BlockSpec((emb, emb), lambda i: (0, 0)),
            pl.BlockSpec((1, emb), lambda i: (0, 0)),
        ],
        out_specs=pl.BlockSpec((block_e, emb), lambda i: (i, 0)),
        out_shape=jax.ShapeDtypeStruct((E, emb), jnp.float32),
        scratch_shapes=[
            pltpu.VMEM((2 * C + emb, emb), jnp.bfloat16),
        ],
    )(cs, cr, rbf, emb_pad, w_s, w_r, w_rbf, b_rbf, w_q, b_out)


def kernel(rbf, charges, differences, senders, receivers,
           embed_table, W_rbf, b_rbf, W_out, b_out):
    del differences
    E, n_rbf = rbf.shape
    emb = embed_table.shape[1]

    charges = charges.astype(jnp.int32)
    senders = senders.astype(jnp.int32)
    receivers = receivers.astype(jnp.int32)

    cs, cr = _sc_charge_gather(charges, senders, receivers)

    block_e = 6400

    C = 128
    emb_pad = jnp.zeros((C, emb), jnp.float32).at[:embed_table.shape[0]].set(
        embed_table)
    w_s = W_out[:emb]
    w_r = W_out[emb:2 * emb]
    w_q = W_out[2 * emb:]

    return _tc_main(rbf.astype(jnp.bfloat16), cs, cr, emb_pad, w_s, w_r,
                    W_rbf.astype(jnp.bfloat16), b_rbf.reshape(1, emb),
                    w_q, b_out.reshape(1, emb), block_e)


# speedup vs baseline: 1.4885x; 1.0305x over previous
import functools

import jax
import jax.numpy as jnp
from jax import lax
from jax.experimental import pallas as pl
from jax.experimental.pallas import tpu as pltpu
from jax.experimental.pallas import tpu_sc as plsc

SILU_GAIN = 1.6765
_NC = 2
_NS = 16


def _act(x):
    return jax.nn.silu(x) * SILU_GAIN


def _sc_charge_gather(charges, senders, receivers):
    N = charges.shape[0]
    E = senders.shape[0]
    nw = _NC * _NS
    per = E // nw
    L = 16
    assert E % nw == 0 and per % L == 0

    mesh = plsc.VectorSubcoreMesh(
        core_axis_name="c", subcore_axis_name="s",
        num_cores=_NC, num_subcores=_NS)

    @functools.partial(
        pl.kernel,
        out_type=(jax.ShapeDtypeStruct((E,), jnp.int32),
                  jax.ShapeDtypeStruct((E,), jnp.int32)),
        mesh=mesh,
        compiler_params=pltpu.CompilerParams(needs_layout_passes=False),
        scratch_types=[
            pltpu.VMEM((N,), jnp.int32),
            pltpu.VMEM((per,), jnp.int32),
            pltpu.VMEM((per,), jnp.int32),
            pltpu.VMEM((per,), jnp.int32),
            pltpu.VMEM((per,), jnp.int32),
            pltpu.SemaphoreType.DMA,
            pltpu.SemaphoreType.DMA,
            pltpu.SemaphoreType.DMA,
        ],
    )
    def k(charges_hbm, senders_hbm, receivers_hbm, cs_hbm, cr_hbm,
          table_v, idx_s, idx_r, out_s, out_r, sem_t, sem_s, sem_r):
        wid = lax.axis_index("s") * _NC + lax.axis_index("c")
        base = wid * per
        sl = pl.ds(base, per)
        cp_t = pltpu.async_copy(charges_hbm, table_v, sem_t)
        cp_s = pltpu.async_copy(senders_hbm.at[sl], idx_s, sem_s)
        cp_r = pltpu.async_copy(receivers_hbm.at[sl], idx_r, sem_r)
        cp_t.wait()

        def gather_loop(idx_v, out_v):
            def body(i, carry):
                s = pl.ds(i * L, L)
                out_v[s] = plsc.load_gather(table_v, [idx_v[s]])
                return carry
            lax.fori_loop(0, per // L, body, 0)

        cp_s.wait()
        gather_loop(idx_s, out_s)
        w_s = pltpu.async_copy(out_s, cs_hbm.at[sl], sem_s)
        cp_r.wait()
        gather_loop(idx_r, out_r)
        w_r = pltpu.async_copy(out_r, cr_hbm.at[sl], sem_r)
        w_s.wait()
        w_r.wait()

    return k(charges, senders, receivers)


def _tc_main(rbf, cs, cr, emb_pad, w_s, w_r, w_rbf, b_rbf, w_q, b_out,
             block_e):
    E, n_rbf = rbf.shape
    emb = w_s.shape[1]
    C = emb_pad.shape[0]
    G = E // block_e
    dn_t = (((0,), (0,)), ((), ()))

    def body(cs_ref, cr_ref, rbf_ref, emb_ref, ws_ref, wr_ref, wrbf_ref,
             brbf_ref, wq_ref, bout_ref, out_ref, tbl_ref):
        @pl.when(pl.program_id(0) == 0)
        def _():
            tbl_ref[0:C, :] = jnp.dot(
                emb_ref[...], ws_ref[...],
                preferred_element_type=jnp.float32).astype(jnp.bfloat16)
            tbl_ref[C:2 * C, :] = jnp.dot(
                emb_ref[...], wr_ref[...],
                preferred_element_type=jnp.float32).astype(jnp.bfloat16)
            tbl_ref[2 * C:, :] = (wq_ref[...] * SILU_GAIN).astype(jnp.bfloat16)

        off = pl.program_id(0) * block_e
        cs = cs_ref[pl.ds(off, block_e)]
        cr = cr_ref[pl.ds(off, block_e)]
        iot = lax.broadcasted_iota(jnp.int32, (block_e, C), 1)
        oh_s = (iot == cs[:, None]).astype(jnp.bfloat16)
        oh_r = (iot == cr[:, None]).astype(jnp.bfloat16)
        z1 = jnp.dot(rbf_ref[...], wrbf_ref[...],
                     preferred_element_type=jnp.float32) + brbf_ref[...]
        r1 = jax.nn.silu(z1).astype(jnp.bfloat16)
        h = jnp.concatenate([oh_s, oh_r, r1], axis=1)
        out_ref[...] = _act(
            jnp.dot(h, tbl_ref[...], preferred_element_type=jnp.float32)
            + bout_ref[...])

    return pl.pallas_call(
        body,
        grid=(G,),
        in_specs=[
            pl.BlockSpec((E,), lambda i: (0,)),
            pl.BlockSpec((E,), lambda i: (0,)),
            pl.BlockSpec((block_e, n_rbf), lambda i: (i, 0)),
            pl.BlockSpec((C, emb), lambda i: (0, 0)),
            pl.BlockSpec((emb, emb), lambda i: (0, 0)),
            pl.BlockSpec((emb, emb), lambda i: (0, 0)),
            pl.BlockSpec((n_rbf, emb), lambda i: (0, 0)),
            pl.BlockSpec((1, emb), lambda i: (0, 0)),
            pl.BlockSpec((emb, emb), lambda i: (0, 0)),
            pl.BlockSpec((1, emb), lambda i: (0, 0)),
        ],
        out_specs=pl.BlockSpec((block_e, emb), lambda i: (i, 0)),
        out_shape=jax.ShapeDtypeStruct((E, emb), jnp.float32),
        scratch_shapes=[
            pltpu.VMEM((2 * C + emb, emb), jnp.bfloat16),
        ],
    )(cs, cr, rbf, emb_pad, w_s, w_r, w_rbf, b_rbf, w_q, b_out)


def kernel(rbf, charges, differences, senders, receivers,
           embed_table, W_rbf, b_rbf, W_out, b_out):
    del differences
    E, n_rbf = rbf.shape
    emb = embed_table.shape[1]

    charges = charges.astype(jnp.int32)
    senders = senders.astype(jnp.int32)
    receivers = receivers.astype(jnp.int32)

    cs, cr = _sc_charge_gather(charges, senders, receivers)

    block_e = 12800

    C = 128
    emb_pad = jnp.zeros((C, emb), jnp.float32).at[:embed_table.shape[0]].set(
        embed_table)
    w_s = W_out[:emb]
    w_r = W_out[emb:2 * emb]
    w_q = W_out[2 * emb:]

    return _tc_main(rbf.astype(jnp.bfloat16), cs, cr, emb_pad, w_s, w_r,
                    W_rbf.astype(jnp.bfloat16), b_rbf.reshape(1, emb),
                    w_q, b_out.reshape(1, emb), block_e)


# speedup vs baseline: 1.4948x; 1.0042x over previous
import functools

import jax
import jax.numpy as jnp
from jax import lax
from jax.experimental import pallas as pl
from jax.experimental.pallas import tpu as pltpu
from jax.experimental.pallas import tpu_sc as plsc

SILU_GAIN = 1.6765
_NC = 2
_NS = 16


def _act(x):
    return jax.nn.silu(x) * SILU_GAIN


def _sc_charge_gather(charges, senders, receivers):
    N = charges.shape[0]
    E = senders.shape[0]
    nw = _NC * _NS
    per = E // nw
    L = 16
    assert E % nw == 0 and per % L == 0

    mesh = plsc.VectorSubcoreMesh(
        core_axis_name="c", subcore_axis_name="s",
        num_cores=_NC, num_subcores=_NS)

    @functools.partial(
        pl.kernel,
        out_type=(jax.ShapeDtypeStruct((E,), jnp.int32),
                  jax.ShapeDtypeStruct((E,), jnp.int32)),
        mesh=mesh,
        compiler_params=pltpu.CompilerParams(needs_layout_passes=False),
        scratch_types=[
            pltpu.VMEM((N,), jnp.int32),
            pltpu.VMEM((per,), jnp.int32),
            pltpu.VMEM((per,), jnp.int32),
            pltpu.VMEM((per,), jnp.int32),
            pltpu.VMEM((per,), jnp.int32),
            pltpu.SemaphoreType.DMA,
            pltpu.SemaphoreType.DMA,
            pltpu.SemaphoreType.DMA,
        ],
    )
    def k(charges_hbm, senders_hbm, receivers_hbm, cs_hbm, cr_hbm,
          table_v, idx_s, idx_r, out_s, out_r, sem_t, sem_s, sem_r):
        wid = lax.axis_index("s") * _NC + lax.axis_index("c")
        base = wid * per
        sl = pl.ds(base, per)
        cp_t = pltpu.async_copy(charges_hbm, table_v, sem_t)
        cp_s = pltpu.async_copy(senders_hbm.at[sl], idx_s, sem_s)
        cp_r = pltpu.async_copy(receivers_hbm.at[sl], idx_r, sem_r)
        cp_t.wait()

        def gather_loop(idx_v, out_v):
            def body(i, carry):
                s = pl.ds(i * L, L)
                out_v[s] = plsc.load_gather(table_v, [idx_v[s]])
                return carry
            lax.fori_loop(0, per // L, body, 0)

        cp_s.wait()
        gather_loop(idx_s, out_s)
        w_s = pltpu.async_copy(out_s, cs_hbm.at[sl], sem_s)
        cp_r.wait()
        gather_loop(idx_r, out_r)
        w_r = pltpu.async_copy(out_r, cr_hbm.at[sl], sem_r)
        w_s.wait()
        w_r.wait()

    return k(charges, senders, receivers)


def _tc_main(rbf, cs, cr, emb_pad, w_s, w_r, w_rbf, b_rbf, w_q, b_out,
             block_e):
    E, n_rbf = rbf.shape
    emb = w_s.shape[1]
    C = emb_pad.shape[0]
    G = E // block_e
    dn_t = (((0,), (0,)), ((), ()))

    def body(cs_ref, cr_ref, rbf_ref, emb_ref, ws_ref, wr_ref, wrbf_ref,
             brbf_ref, wq_ref, bout_ref, out_ref, tbl_ref):
        @pl.when(pl.program_id(0) == 0)
        def _():
            tbl_ref[0:C, :] = jnp.dot(
                emb_ref[...], ws_ref[...],
                preferred_element_type=jnp.float32).astype(jnp.bfloat16)
            tbl_ref[C:2 * C, :] = jnp.dot(
                emb_ref[...], wr_ref[...],
                preferred_element_type=jnp.float32).astype(jnp.bfloat16)
            tbl_ref[2 * C:, :] = (wq_ref[...] * SILU_GAIN).astype(jnp.bfloat16)

        off = pl.program_id(0) * block_e
        cs = cs_ref[pl.ds(off, block_e)]
        cr = cr_ref[pl.ds(off, block_e)]
        iot = lax.broadcasted_iota(jnp.int32, (block_e, C), 1)
        oh_s = (iot == cs[:, None]).astype(jnp.bfloat16)
        oh_r = (iot == cr[:, None]).astype(jnp.bfloat16)
        z1 = jnp.dot(rbf_ref[...], wrbf_ref[...],
                     preferred_element_type=jnp.float32) + brbf_ref[...]
        r1 = jax.nn.silu(z1).astype(jnp.bfloat16)
        h = jnp.concatenate([oh_s, oh_r, r1], axis=1)
        out_ref[...] = _act(
            jnp.dot(h, tbl_ref[...], preferred_element_type=jnp.float32)
            + bout_ref[...])

    return pl.pallas_call(
        body,
        grid=(G,),
        in_specs=[
            pl.BlockSpec((E,), lambda i: (0,)),
            pl.BlockSpec((E,), lambda i: (0,)),
            pl.BlockSpec((block_e, n_rbf), lambda i: (i, 0)),
            pl.BlockSpec((C, emb), lambda i: (0, 0)),
            pl.BlockSpec((emb, emb), lambda i: (0, 0)),
            pl.BlockSpec((emb, emb), lambda i: (0, 0)),
            pl.BlockSpec((n_rbf, emb), lambda i: (0, 0)),
            pl.BlockSpec((1, emb), lambda i: (0, 0)),
            pl.BlockSpec((emb, emb), lambda i: (0, 0)),
            pl.BlockSpec((1, emb), lambda i: (0, 0)),
        ],
        out_specs=pl.BlockSpec((block_e, emb), lambda i: (i, 0)),
        out_shape=jax.ShapeDtypeStruct((E, emb), jnp.float32),
        scratch_shapes=[
            pltpu.VMEM((2 * C + emb, emb), jnp.bfloat16),
        ],
    )(cs, cr, rbf, emb_pad, w_s, w_r, w_rbf, b_rbf, w_q, b_out)


def kernel(rbf, charges, differences, senders, receivers,
           embed_table, W_rbf, b_rbf, W_out, b_out):
    del differences
    E, n_rbf = rbf.shape
    emb = embed_table.shape[1]

    charges = charges.astype(jnp.int32)
    senders = senders.astype(jnp.int32)
    receivers = receivers.astype(jnp.int32)

    cs, cr = _sc_charge_gather(charges, senders, receivers)

    block_e = 16000

    C = 128
    emb_pad = jnp.zeros((C, emb), jnp.float32).at[:embed_table.shape[0]].set(
        embed_table)
    w_s = W_out[:emb]
    w_r = W_out[emb:2 * emb]
    w_q = W_out[2 * emb:]

    return _tc_main(rbf.astype(jnp.bfloat16), cs, cr, emb_pad, w_s, w_r,
                    W_rbf.astype(jnp.bfloat16), b_rbf.reshape(1, emb),
                    w_q, b_out.reshape(1, emb), block_e)


# speedup vs baseline: 1.8340x; 1.2269x over previous
import functools

import jax
import jax.numpy as jnp
from jax import lax
from jax.experimental import pallas as pl
from jax.experimental.pallas import tpu as pltpu
from jax.experimental.pallas import tpu_sc as plsc

SILU_GAIN = 1.6765
_NC = 2
_NS = 16


def _act(x):
    return jax.nn.silu(x) * SILU_GAIN


def _sc_charge_gather(charges, senders, receivers):
    N = charges.shape[0]
    E = senders.shape[0]
    nw = _NC * _NS
    per = E // nw
    L = 16
    assert E % nw == 0 and per % L == 0

    mesh = plsc.VectorSubcoreMesh(
        core_axis_name="c", subcore_axis_name="s",
        num_cores=_NC, num_subcores=_NS)

    @functools.partial(
        pl.kernel,
        out_type=(jax.ShapeDtypeStruct((E,), jnp.int32),
                  jax.ShapeDtypeStruct((E,), jnp.int32)),
        mesh=mesh,
        compiler_params=pltpu.CompilerParams(needs_layout_passes=False),
        scratch_types=[
            pltpu.VMEM((N,), jnp.int32),
            pltpu.VMEM((per,), jnp.int32),
            pltpu.VMEM((per,), jnp.int32),
            pltpu.VMEM((per,), jnp.int32),
            pltpu.VMEM((per,), jnp.int32),
            pltpu.SemaphoreType.DMA,
            pltpu.SemaphoreType.DMA,
            pltpu.SemaphoreType.DMA,
        ],
    )
    def k(charges_hbm, senders_hbm, receivers_hbm, cs_hbm, cr_hbm,
          table_v, idx_s, idx_r, out_s, out_r, sem_t, sem_s, sem_r):
        wid = lax.axis_index("s") * _NC + lax.axis_index("c")
        base = wid * per
        sl = pl.ds(base, per)
        cp_t = pltpu.async_copy(charges_hbm, table_v, sem_t)
        cp_s = pltpu.async_copy(senders_hbm.at[sl], idx_s, sem_s)
        cp_r = pltpu.async_copy(receivers_hbm.at[sl], idx_r, sem_r)
        cp_t.wait()

        def gather_loop(idx_v, out_v):
            def body(i, carry):
                s = pl.ds(i * L, L)
                out_v[s] = plsc.load_gather(table_v, [idx_v[s]])
                return carry
            lax.fori_loop(0, per // L, body, 0)

        cp_s.wait()
        gather_loop(idx_s, out_s)
        w_s = pltpu.async_copy(out_s, cs_hbm.at[sl], sem_s)
        cp_r.wait()
        gather_loop(idx_r, out_r)
        w_r = pltpu.async_copy(out_r, cr_hbm.at[sl], sem_r)
        w_s.wait()
        w_r.wait()

    return k(charges, senders, receivers)


def _tc_main(rbf_t, cs, cr, emb_pad, w_s, w_r, w_rbf, b_rbf, w_q, b_out,
             block_e):
    n_rbf, E = rbf_t.shape
    emb = w_s.shape[1]
    C = emb_pad.shape[0]
    G = E // block_e
    dn_t = (((0,), (0,)), ((), ()))

    def body(cs_ref, cr_ref, rbf_ref, emb_ref, ws_ref, wr_ref, wrbf_ref,
             brbf_ref, wq_ref, bout_ref, out_ref, tbl_ref):
        @pl.when(pl.program_id(0) == 0)
        def _():
            tbl_ref[0:C, :] = jnp.dot(
                emb_ref[...], ws_ref[...],
                preferred_element_type=jnp.float32).astype(jnp.bfloat16)
            tbl_ref[C:2 * C, :] = jnp.dot(
                emb_ref[...], wr_ref[...],
                preferred_element_type=jnp.float32).astype(jnp.bfloat16)
            tbl_ref[2 * C:, :] = (wq_ref[...] * SILU_GAIN).astype(jnp.bfloat16)

        off = pl.program_id(0) * block_e
        cs = cs_ref[pl.ds(off, block_e)]
        cr = cr_ref[pl.ds(off, block_e)]
        iot = lax.broadcasted_iota(jnp.int32, (block_e, C), 1)
        oh_s = (iot == cs[:, None]).astype(jnp.bfloat16)
        oh_r = (iot == cr[:, None]).astype(jnp.bfloat16)
        z1 = lax.dot_general(rbf_ref[...], wrbf_ref[...], dn_t,
                             preferred_element_type=jnp.float32) + brbf_ref[...]
        r1 = jax.nn.silu(z1).astype(jnp.bfloat16)
        h = jnp.concatenate([oh_s, oh_r, r1], axis=1)
        out_ref[...] = _act(
            jnp.dot(h, tbl_ref[...], preferred_element_type=jnp.float32)
            + bout_ref[...])

    return pl.pallas_call(
        body,
        grid=(G,),
        compiler_params=pltpu.CompilerParams(
            fuse_transposed_lhs_in_matmul=True),
        in_specs=[
            pl.BlockSpec((E,), lambda i: (0,)),
            pl.BlockSpec((E,), lambda i: (0,)),
            pl.BlockSpec((n_rbf, block_e), lambda i: (0, i)),
            pl.BlockSpec((C, emb), lambda i: (0, 0)),
            pl.BlockSpec((emb, emb), lambda i: (0, 0)),
            pl.BlockSpec((emb, emb), lambda i: (0, 0)),
            pl.BlockSpec((n_rbf, emb), lambda i: (0, 0)),
            pl.BlockSpec((1, emb), lambda i: (0, 0)),
            pl.BlockSpec((emb, emb), lambda i: (0, 0)),
            pl.BlockSpec((1, emb), lambda i: (0, 0)),
        ],
        out_specs=pl.BlockSpec((block_e, emb), lambda i: (i, 0)),
        out_shape=jax.ShapeDtypeStruct((E, emb), jnp.float32),
        scratch_shapes=[
            pltpu.VMEM((2 * C + emb, emb), jnp.bfloat16),
        ],
    )(cs, cr, rbf_t, emb_pad, w_s, w_r, w_rbf, b_rbf, w_q, b_out)


def kernel(rbf, charges, differences, senders, receivers,
           embed_table, W_rbf, b_rbf, W_out, b_out):
    del differences
    E, n_rbf = rbf.shape
    emb = embed_table.shape[1]

    charges = charges.astype(jnp.int32)
    senders = senders.astype(jnp.int32)
    receivers = receivers.astype(jnp.int32)

    cs, cr = _sc_charge_gather(charges, senders, receivers)

    block_e = 16000

    C = 128
    emb_pad = jnp.zeros((C, emb), jnp.float32).at[:embed_table.shape[0]].set(
        embed_table)
    w_s = W_out[:emb]
    w_r = W_out[emb:2 * emb]
    w_q = W_out[2 * emb:]

    rbf_t = rbf.T.astype(jnp.bfloat16)
    return _tc_main(rbf_t, cs, cr, emb_pad, w_s, w_r,
                    W_rbf.astype(jnp.bfloat16), b_rbf.reshape(1, emb),
                    w_q, b_out.reshape(1, emb), block_e)


# speedup vs baseline: 1.9259x; 1.0501x over previous
import functools

import jax
import jax.numpy as jnp
from jax import lax
from jax.experimental import pallas as pl
from jax.experimental.pallas import tpu as pltpu
from jax.experimental.pallas import tpu_sc as plsc

SILU_GAIN = 1.6765
_NC = 2
_NS = 16


def _act(x):
    return jax.nn.silu(x) * SILU_GAIN


def _silu_tanh(x):
    return x * (0.5 * jnp.tanh(0.5 * x) + 0.5)


def _sc_charge_gather(charges, senders, receivers):
    N = charges.shape[0]
    E = senders.shape[0]
    nw = _NC * _NS
    per = E // nw
    L = 16
    assert E % nw == 0 and per % L == 0

    mesh = plsc.VectorSubcoreMesh(
        core_axis_name="c", subcore_axis_name="s",
        num_cores=_NC, num_subcores=_NS)

    @functools.partial(
        pl.kernel,
        out_type=(jax.ShapeDtypeStruct((E,), jnp.int32),
                  jax.ShapeDtypeStruct((E,), jnp.int32)),
        mesh=mesh,
        compiler_params=pltpu.CompilerParams(needs_layout_passes=False),
        scratch_types=[
            pltpu.VMEM((N,), jnp.int32),
            pltpu.VMEM((per,), jnp.int32),
            pltpu.VMEM((per,), jnp.int32),
            pltpu.VMEM((per,), jnp.int32),
            pltpu.VMEM((per,), jnp.int32),
            pltpu.SemaphoreType.DMA,
            pltpu.SemaphoreType.DMA,
            pltpu.SemaphoreType.DMA,
        ],
    )
    def k(charges_hbm, senders_hbm, receivers_hbm, cs_hbm, cr_hbm,
          table_v, idx_s, idx_r, out_s, out_r, sem_t, sem_s, sem_r):
        wid = lax.axis_index("s") * _NC + lax.axis_index("c")
        base = wid * per
        sl = pl.ds(base, per)
        cp_t = pltpu.async_copy(charges_hbm, table_v, sem_t)
        cp_s = pltpu.async_copy(senders_hbm.at[sl], idx_s, sem_s)
        cp_r = pltpu.async_copy(receivers_hbm.at[sl], idx_r, sem_r)
        cp_t.wait()

        def gather_loop(idx_v, out_v):
            def body(i, carry):
                s = pl.ds(i * L, L)
                out_v[s] = plsc.load_gather(table_v, [idx_v[s]])
                return carry
            lax.fori_loop(0, per // L, body, 0)

        cp_s.wait()
        gather_loop(idx_s, out_s)
        w_s = pltpu.async_copy(out_s, cs_hbm.at[sl], sem_s)
        cp_r.wait()
        gather_loop(idx_r, out_r)
        w_r = pltpu.async_copy(out_r, cr_hbm.at[sl], sem_r)
        w_s.wait()
        w_r.wait()

    return k(charges, senders, receivers)


def _tc_main(rbf_t, cs, cr, emb_pad, w_s, w_r, w_rbf, b_rbf, w_q, b_out,
             block_e):
    n_rbf, E = rbf_t.shape
    emb = w_s.shape[1]
    C = emb_pad.shape[0]
    G = E // block_e
    dn_t = (((0,), (0,)), ((), ()))

    def body(cs_ref, cr_ref, rbf_ref, emb_ref, ws_ref, wr_ref, wrbf_ref,
             brbf_ref, wq_ref, bout_ref, out_ref, tbl_ref):
        @pl.when(pl.program_id(0) == 0)
        def _():
            tbl_ref[0:C, :] = jnp.dot(
                emb_ref[...], ws_ref[...],
                preferred_element_type=jnp.float32).astype(jnp.bfloat16)
            tbl_ref[C:2 * C, :] = jnp.dot(
                emb_ref[...], wr_ref[...],
                preferred_element_type=jnp.float32).astype(jnp.bfloat16)
            tbl_ref[2 * C:, :] = (wq_ref[...] * SILU_GAIN).astype(jnp.bfloat16)

        off = pl.program_id(0) * block_e
        cs = cs_ref[pl.ds(off, block_e)]
        cr = cr_ref[pl.ds(off, block_e)]
        iot = lax.broadcasted_iota(jnp.int32, (block_e, C), 1)
        oh_s = (iot == cs[:, None]).astype(jnp.bfloat16)
        oh_r = (iot == cr[:, None]).astype(jnp.bfloat16)
        z1 = lax.dot_general(rbf_ref[...], wrbf_ref[...], dn_t,
                             preferred_element_type=jnp.float32) + brbf_ref[...]
        r1 = _silu_tanh(z1).astype(jnp.bfloat16)
        h = jnp.concatenate([oh_s, oh_r, r1], axis=1)
        z = (jnp.dot(h, tbl_ref[...], preferred_element_type=jnp.float32)
             + bout_ref[...])
        out_ref[...] = _silu_tanh(z) * SILU_GAIN

    return pl.pallas_call(
        body,
        grid=(G,),
        compiler_params=pltpu.CompilerParams(
            fuse_transposed_lhs_in_matmul=True),
        in_specs=[
            pl.BlockSpec((E,), lambda i: (0,)),
            pl.BlockSpec((E,), lambda i: (0,)),
            pl.BlockSpec((n_rbf, block_e), lambda i: (0, i)),
            pl.BlockSpec((C, emb), lambda i: (0, 0)),
            pl.BlockSpec((emb, emb), lambda i: (0, 0)),
            pl.BlockSpec((emb, emb), lambda i: (0, 0)),
            pl.BlockSpec((n_rbf, emb), lambda i: (0, 0)),
            pl.BlockSpec((1, emb), lambda i: (0, 0)),
            pl.BlockSpec((emb, emb), lambda i: (0, 0)),
            pl.BlockSpec((1, emb), lambda i: (0, 0)),
        ],
        out_specs=pl.BlockSpec((block_e, emb), lambda i: (i, 0)),
        out_shape=jax.ShapeDtypeStruct((E, emb), jnp.float32),
        scratch_shapes=[
            pltpu.VMEM((2 * C + emb, emb), jnp.bfloat16),
        ],
    )(cs, cr, rbf_t, emb_pad, w_s, w_r, w_rbf, b_rbf, w_q, b_out)


def kernel(rbf, charges, differences, senders, receivers,
           embed_table, W_rbf, b_rbf, W_out, b_out):
    del differences
    E, n_rbf = rbf.shape
    emb = embed_table.shape[1]

    charges = charges.astype(jnp.int32)
    senders = senders.astype(jnp.int32)
    receivers = receivers.astype(jnp.int32)

    cs, cr = _sc_charge_gather(charges, senders, receivers)

    block_e = 16000

    C = 128
    emb_pad = jnp.zeros((C, emb), jnp.float32).at[:embed_table.shape[0]].set(
        embed_table)
    w_s = W_out[:emb]
    w_r = W_out[emb:2 * emb]
    w_q = W_out[2 * emb:]

    rbf_t = rbf.T.astype(jnp.bfloat16)
    return _tc_main(rbf_t, cs, cr, emb_pad, w_s, w_r,
                    W_rbf.astype(jnp.bfloat16), b_rbf.reshape(1, emb),
                    w_q, b_out.reshape(1, emb), block_e)
